# baseline (speedup 1.0000x reference)
import functools

import jax
import jax.numpy as jnp
from jax import lax
from jax.experimental import pallas as pl
from jax.experimental.pallas import tpu as pltpu
from jax.experimental.pallas import tpu_sc as plsc

GEO_DIM = 16
TEMPORAL_DIM = 4
OUT_DIM = 32

SLAB_COLS = 2560
CAP = 768
LIST = CAP + 16


@functools.lru_cache(maxsize=None)
def _make_lookup(V, D, B):
  info = plsc.get_sparse_core_info()
  NC, NS = info.num_cores, info.num_subcores
  NW = NC * NS
  n_slabs = (V + SLAB_COLS - 1) // SLAB_COLS
  n_rounds = (n_slabs + NW - 1) // NW
  last_slab = n_slabs - 1
  tail_cols = V - last_slab * SLAB_COLS
  n_groups = B // 16
  mesh = plsc.VectorSubcoreMesh(core_axis_name="c", subcore_axis_name="s")

  @functools.partial(
      pl.kernel,
      mesh=mesh,
      compiler_params=pltpu.CompilerParams(
          use_tc_tiling_on_sc=True, needs_layout_passes=False),
      out_type=(jax.ShapeDtypeStruct((NW * CAP * D,), jnp.float32),
                jax.ShapeDtypeStruct((NW * CAP,), jnp.int32)),
      scratch_types=[
          pltpu.VMEM((B,), jnp.int32),
          pltpu.VMEM((LIST,), jnp.int32),
          pltpu.VMEM((LIST,), jnp.int32),
          pltpu.VMEM((LIST,), jnp.int32),
          pltpu.VMEM((LIST,), jnp.int32),
          pltpu.VMEM(((CAP + 16) * D,), jnp.float32),
          pltpu.VMEM((D, SLAB_COLS), jnp.float32),
          pltpu.VMEM((D, SLAB_COLS), jnp.float32),
          pltpu.SemaphoreType.DMA,
      ],
  )
  def lookup_k(table_hbm, tail_hbm, idx_hbm, rows_hbm, pos_hbm,
               idx_v, myidx_v, mypos_v, gidx_v, gpos_v,
               stage_v, slab_a, slab_b, sem):
    w = lax.axis_index("s") * NC + lax.axis_index("c")
    d16 = lax.iota(jnp.int32, 16)
    slabs = (slab_a, slab_b)

    cps = [pltpu.async_copy(table_hbm.at[:, pl.ds(w * SLAB_COLS, SLAB_COLS)],
                            slab_a, sem)]

    pltpu.sync_copy(idx_hbm, idx_v)

    def scan_body(k, cur):
      v = idx_v[pl.ds(k * 16, 16)]
      mine = ((v // SLAB_COLS) & (NW - 1)) == w
      plsc.store_compressed(myidx_v.at[pl.ds(cur, 16)], v, mask=mine)
      plsc.store_compressed(mypos_v.at[pl.ds(cur, 16)], k * 16 + d16, mask=mine)
      cnt = plsc.all_reduce_population_count(mine)[0]
      return jnp.minimum(cur + cnt, CAP)

    cur = lax.fori_loop(0, n_groups, scan_body, jnp.int32(0), unroll=4)

    for i in range(LIST // 16):
      gpos_v[pl.ds(i * 16, 16)] = jnp.full((16,), B, dtype=jnp.int32)

    n_win = (cur + 15) >> 4
    starts = []
    cur2 = jnp.int32(0)
    for r in range(n_rounds):
      starts.append(cur2)

      def group_body(k, c2, r=r):
        e = myidx_v[pl.ds(k * 16, 16)]
        p = mypos_v[pl.ds(k * 16, 16)]
        m = ((e // (SLAB_COLS * NW)) == r) & ((k * 16 + d16) < cur)
        plsc.store_compressed(gidx_v.at[pl.ds(c2, 16)], e, mask=m)
        plsc.store_compressed(gpos_v.at[pl.ds(c2, 16)], p, mask=m)
        cnt = plsc.all_reduce_population_count(m)[0]
        return jnp.minimum(c2 + cnt, CAP)

      cur2 = lax.fori_loop(0, n_win, group_body, cur2)
    starts.append(cur2)

    def gather_round(r, buf, tail):
      base_c = (r * NW + w) * SLAB_COLS

      def gather_body(g, _):
        base_e = starts[r] + g * 16
        colv = gidx_v[pl.ds(base_e, 16)] - base_c
        for j in range(16):
          @pl.when(base_e + j < starts[r + 1])
          def _():
            cj = colv[j]
            row = plsc.load_gather(buf, [d16, jnp.full((16,), cj)])
            plsc.store_scatter(stage_v, [(base_e + j) * D + d16], row)
        return 0

      n_g = (starts[r + 1] - starts[r] + 15) >> 4
      lax.fori_loop(0, n_g, gather_body, 0)

    for r in range(n_rounds - 1):
      cps[r].wait()
      if r + 1 <= n_rounds - 2:
        nxt = (r + 1) * NW + w
        cps.append(
            pltpu.async_copy(table_hbm.at[:, pl.ds(nxt * SLAB_COLS, SLAB_COLS)],
                             slabs[(r + 1) % 2], sem))
      gather_round(r, slabs[r % 2], tail=False)

    last_buf = slabs[(n_rounds - 1) % 2]

    @pl.when(w < (n_slabs - (n_rounds - 1) * NW) - 1)
    def _():
      base_c = ((n_rounds - 1) * NW + w) * SLAB_COLS
      pltpu.sync_copy(table_hbm.at[:, pl.ds(base_c, SLAB_COLS)],
                      last_buf.at[:, pl.ds(0, SLAB_COLS)])
      gather_round(n_rounds - 1, last_buf, tail=False)

    @pl.when(w == (n_slabs - (n_rounds - 1) * NW) - 1)
    def _():
      pltpu.sync_copy(tail_hbm, last_buf.at[:, pl.ds(0, tail_hbm.shape[1])])
      gather_round(n_rounds - 1, last_buf, tail=True)

    pltpu.sync_copy(stage_v.at[pl.ds(0, CAP * D)],
                    rows_hbm.at[pl.ds(w * CAP * D, CAP * D)])
    pltpu.sync_copy(gpos_v.at[pl.ds(0, CAP)], pos_hbm.at[pl.ds(w * CAP, CAP)])

  return lookup_k


@functools.lru_cache(maxsize=None)
def _make_scatter(D, B):
  info = plsc.get_sparse_core_info()
  NC, NS = info.num_cores, info.num_subcores
  NW = NC * NS
  n_win = CAP // 128
  mesh = plsc.VectorSubcoreMesh(core_axis_name="c", subcore_axis_name="s")

  seg = B // NW

  @functools.partial(
      pl.kernel,
      mesh=mesh,
      compiler_params=pltpu.CompilerParams(
          use_tc_tiling_on_sc=False, needs_layout_passes=False),
      out_type=jax.ShapeDtypeStruct((B, D), jnp.float32),
      scratch_types=[
          pltpu.VMEM((NW * CAP,), jnp.int32),
          pltpu.VMEM((seg // 128, 128), jnp.int32),
          pltpu.VMEM((seg, D), jnp.float32),
          pltpu.SemaphoreType.DMA,
      ],
  )
  def unpermute_k(rows_hbm, pos_hbm, out_hbm, pos_v, e2p_v, rows_v, sem):
    w = lax.axis_index("s") * NC + lax.axis_index("c")
    d16 = lax.iota(jnp.int32, 16)
    pltpu.sync_copy(pos_hbm, pos_v)

    def scan_body(k, _):
      p = pos_v[pl.ds(k * 16, 16)]
      m = (p >> 9) == w
      off = p & (seg - 1)
      plsc.store_scatter(e2p_v, [off >> 7, off & 127], k * 16 + d16, mask=m)
      return 0

    lax.fori_loop(0, (NW * CAP) // 16, scan_body, 0, unroll=8)

    copies = []
    for q in range(seg // 128):
      copies.append(
          pltpu.async_copy(rows_hbm.at[e2p_v.at[q]],
                           rows_v.at[pl.ds(q * 128, 128)], sem))
    for c in copies:
      c.wait()
    pltpu.sync_copy(rows_v, out_hbm.at[pl.ds(w * seg, seg)])

  return unpermute_k


def _dense_body(t_ref, g_ref, wt_ref, wg_ref, b_ref, ga_ref, be_ref, o_ref):
  dn = (((1,), (1,)), ((), ()))
  h = lax.dot_general(wt_ref[...], t_ref[...], dn,
                      preferred_element_type=jnp.float32)
  h = h + lax.dot_general(wg_ref[...], g_ref[...], dn,
                          preferred_element_type=jnp.float32)
  h = h + b_ref[...]
  mean = jnp.mean(h, axis=0, keepdims=True)
  c = h - mean
  var = jnp.mean(c * c, axis=0, keepdims=True)
  normed = c * lax.rsqrt(var + 1e-5)
  normed = normed * ga_ref[...] + be_ref[...]
  o_ref[...] = normed * 0.5 * (1.0 + lax.erf(normed * (2.0 ** -0.5)))


def _dense(temporal, geo, wtT, wgT, b2, ga2, be2, block_b=4096, interpret=False):
  B = temporal.shape[0]
  grid = (B // block_b,)
  out = pl.pallas_call(
      _dense_body,
      grid=grid,
      in_specs=[
          pl.BlockSpec((block_b, TEMPORAL_DIM), lambda i: (i, 0)),
          pl.BlockSpec((block_b, GEO_DIM), lambda i: (i, 0)),
          pl.BlockSpec((OUT_DIM, TEMPORAL_DIM), lambda i: (0, 0)),
          pl.BlockSpec((OUT_DIM, GEO_DIM), lambda i: (0, 0)),
          pl.BlockSpec((OUT_DIM, 1), lambda i: (0, 0)),
          pl.BlockSpec((OUT_DIM, 1), lambda i: (0, 0)),
          pl.BlockSpec((OUT_DIM, 1), lambda i: (0, 0)),
      ],
      out_specs=pl.BlockSpec((OUT_DIM, block_b), lambda i: (0, i)),
      out_shape=jax.ShapeDtypeStruct((OUT_DIM, B), jnp.float32),
      interpret=interpret,
  )(temporal, geo, wtT, wgT, b2, ga2, be2)
  return out.T


def kernel(temporal_features, geohash_buckets, emb_table, W, b, ln_gamma, ln_beta):
  B = temporal_features.shape[0]
  V, D = emb_table.shape
  idx = geohash_buckets.astype(jnp.int32)
  tbl_t = emb_table.T
  tail_start = (V - 1) // SLAB_COLS * SLAB_COLS
  tail_cols = V - tail_start
  tail_pad = (tail_cols + 127) // 128 * 128
  tail = jnp.pad(emb_table[tail_start:].T, ((0, 0), (0, tail_pad - tail_cols)))
  rows, pos = _make_lookup(V, D, B)(tbl_t, tail, idx)
  geo = _make_scatter(D, B)(rows.reshape(-1, D), pos)
  wtT = W[:TEMPORAL_DIM].T
  wgT = W[TEMPORAL_DIM:].T
  return _dense(temporal_features, geo, wtT, wgT,
                b[:, None], ln_gamma[:, None], ln_beta[:, None])


# speedup vs baseline: 1.4958x; 1.4958x over previous
import functools

import jax
import jax.numpy as jnp
from jax import lax
from jax.experimental import pallas as pl
from jax.experimental.pallas import tpu as pltpu
from jax.experimental.pallas import tpu_sc as plsc

GEO_DIM = 16
TEMPORAL_DIM = 4
OUT_DIM = 32

SLAB_COLS = 2048
CAP = 768
LIST = CAP + 16


@functools.lru_cache(maxsize=None)
def _make_lookup(V, D, B):
  info = plsc.get_sparse_core_info()
  NC, NS = info.num_cores, info.num_subcores
  NW = NC * NS
  n_slabs = (V + SLAB_COLS - 1) // SLAB_COLS
  n_rounds = (n_slabs + NW - 1) // NW
  last_slab = n_slabs - 1
  tail_cols = V - last_slab * SLAB_COLS
  n_groups = B // 16
  mesh = plsc.VectorSubcoreMesh(core_axis_name="c", subcore_axis_name="s")

  @functools.partial(
      pl.kernel,
      mesh=mesh,
      compiler_params=pltpu.CompilerParams(
          use_tc_tiling_on_sc=True, needs_layout_passes=False),
      out_type=(jax.ShapeDtypeStruct((NW * CAP * D,), jnp.float32),
                jax.ShapeDtypeStruct((NW * CAP,), jnp.int32)),
      scratch_types=[
          pltpu.VMEM((B,), jnp.int32),
          pltpu.VMEM((LIST,), jnp.int32),
          pltpu.VMEM((LIST,), jnp.int32),
          pltpu.VMEM((LIST,), jnp.int32),
          pltpu.VMEM((LIST,), jnp.int32),
          pltpu.VMEM(((CAP + 16) * D,), jnp.float32),
          pltpu.VMEM((D, SLAB_COLS), jnp.float32),
          pltpu.VMEM((D, SLAB_COLS), jnp.float32),
          pltpu.SemaphoreType.DMA,
      ],
  )
  def lookup_k(table_hbm, tail_hbm, idx_hbm, rows_hbm, pos_hbm,
               idx_v, myidx_v, mypos_v, gidx_v, gpos_v,
               stage_v, slab_a, slab_b, sem):
    w = lax.axis_index("s") * NC + lax.axis_index("c")
    d16 = lax.iota(jnp.int32, 16)
    slabs = (slab_a, slab_b)

    cps = [pltpu.async_copy(table_hbm.at[:, pl.ds(w * SLAB_COLS, SLAB_COLS)],
                            slab_a, sem)]

    pltpu.sync_copy(idx_hbm, idx_v)

    def scan_body(k, cur):
      v = idx_v[pl.ds(k * 16, 16)]
      mine = ((v >> 11) & (NW - 1)) == w
      plsc.store_compressed(myidx_v.at[pl.ds(cur, 16)], v, mask=mine)
      plsc.store_compressed(mypos_v.at[pl.ds(cur, 16)], k * 16 + d16, mask=mine)
      cnt = plsc.all_reduce_population_count(mine)[0]
      return jnp.minimum(cur + cnt, CAP)

    cur = lax.fori_loop(0, n_groups, scan_body, jnp.int32(0), unroll=4)

    for i in range(LIST // 16):
      gpos_v[pl.ds(i * 16, 16)] = jnp.full((16,), B, dtype=jnp.int32)

    n_win = (cur + 15) >> 4
    starts = []
    cur2 = jnp.int32(0)
    for r in range(n_rounds):
      starts.append(cur2)

      def group_body(k, c2, r=r):
        e = myidx_v[pl.ds(k * 16, 16)]
        p = mypos_v[pl.ds(k * 16, 16)]
        m = ((e >> 16) == r) & ((k * 16 + d16) < cur)
        plsc.store_compressed(gidx_v.at[pl.ds(c2, 16)], e, mask=m)
        plsc.store_compressed(gpos_v.at[pl.ds(c2, 16)], p, mask=m)
        cnt = plsc.all_reduce_population_count(m)[0]
        return jnp.minimum(c2 + cnt, CAP)

      cur2 = lax.fori_loop(0, n_win, group_body, cur2)
    starts.append(cur2)

    def gather_round(r, buf, tail):
      base_c = (r * NW + w) * SLAB_COLS

      def gather_body(g, _):
        base_e = starts[r] + g * 16
        colv = gidx_v[pl.ds(base_e, 16)] - base_c
        for j in range(16):
          @pl.when(base_e + j < starts[r + 1])
          def _():
            cj = colv[j]
            row = plsc.load_gather(buf, [d16, jnp.full((16,), cj)])
            plsc.store_scatter(stage_v, [(base_e + j) * D + d16], row)
        return 0

      n_g = (starts[r + 1] - starts[r] + 15) >> 4
      lax.fori_loop(0, n_g, gather_body, 0)

    for r in range(n_rounds - 1):
      cps[r].wait()
      if r + 1 <= n_rounds - 2:
        nxt = (r + 1) * NW + w
        cps.append(
            pltpu.async_copy(table_hbm.at[:, pl.ds(nxt * SLAB_COLS, SLAB_COLS)],
                             slabs[(r + 1) % 2], sem))
      gather_round(r, slabs[r % 2], tail=False)

    last_buf = slabs[(n_rounds - 1) % 2]

    @pl.when(w < (n_slabs - (n_rounds - 1) * NW) - 1)
    def _():
      base_c = ((n_rounds - 1) * NW + w) * SLAB_COLS
      pltpu.sync_copy(table_hbm.at[:, pl.ds(base_c, SLAB_COLS)],
                      last_buf.at[:, pl.ds(0, SLAB_COLS)])
      gather_round(n_rounds - 1, last_buf, tail=False)

    @pl.when(w == (n_slabs - (n_rounds - 1) * NW) - 1)
    def _():
      pltpu.sync_copy(tail_hbm, last_buf.at[:, pl.ds(0, tail_hbm.shape[1])])
      gather_round(n_rounds - 1, last_buf, tail=True)

    pltpu.sync_copy(stage_v.at[pl.ds(0, CAP * D)],
                    rows_hbm.at[pl.ds(w * CAP * D, CAP * D)])
    pltpu.sync_copy(gpos_v.at[pl.ds(0, CAP)], pos_hbm.at[pl.ds(w * CAP, CAP)])

  return lookup_k


@functools.lru_cache(maxsize=None)
def _make_scatter(D, B):
  info = plsc.get_sparse_core_info()
  NC, NS = info.num_cores, info.num_subcores
  NW = NC * NS
  n_win = CAP // 128
  mesh = plsc.VectorSubcoreMesh(core_axis_name="c", subcore_axis_name="s")

  seg = B // NW

  @functools.partial(
      pl.kernel,
      mesh=mesh,
      compiler_params=pltpu.CompilerParams(
          use_tc_tiling_on_sc=False, needs_layout_passes=False),
      out_type=jax.ShapeDtypeStruct((B, D), jnp.float32),
      scratch_types=[
          pltpu.VMEM((NW * CAP,), jnp.int32),
          pltpu.VMEM((seg // 128, 128), jnp.int32),
          pltpu.VMEM((seg, D), jnp.float32),
          pltpu.SemaphoreType.DMA,
      ],
  )
  def unpermute_k(rows_hbm, pos_hbm, out_hbm, pos_v, e2p_v, rows_v, sem):
    w = lax.axis_index("s") * NC + lax.axis_index("c")
    d16 = lax.iota(jnp.int32, 16)
    pltpu.sync_copy(pos_hbm, pos_v)

    def scan_body(k, _):
      p = pos_v[pl.ds(k * 16, 16)]
      m = (p >> 9) == w
      off = p & (seg - 1)
      plsc.store_scatter(e2p_v, [off >> 7, off & 127], k * 16 + d16, mask=m)
      return 0

    lax.fori_loop(0, (NW * CAP) // 16, scan_body, 0, unroll=8)

    copies = []
    for q in range(seg // 128):
      copies.append(
          pltpu.async_copy(rows_hbm.at[e2p_v.at[q]],
                           rows_v.at[pl.ds(q * 128, 128)], sem))
    for c in copies:
      c.wait()
    pltpu.sync_copy(rows_v, out_hbm.at[pl.ds(w * seg, seg)])

  return unpermute_k


def _dense_body(t_ref, g_ref, wt_ref, wg_ref, b_ref, ga_ref, be_ref, o_ref):
  dn = (((1,), (1,)), ((), ()))
  h = lax.dot_general(wt_ref[...], t_ref[...], dn,
                      preferred_element_type=jnp.float32)
  h = h + lax.dot_general(wg_ref[...], g_ref[...], dn,
                          preferred_element_type=jnp.float32)
  h = h + b_ref[...]
  mean = jnp.mean(h, axis=0, keepdims=True)
  c = h - mean
  var = jnp.mean(c * c, axis=0, keepdims=True)
  normed = c * lax.rsqrt(var + 1e-5)
  normed = normed * ga_ref[...] + be_ref[...]
  o_ref[...] = normed * 0.5 * (1.0 + lax.erf(normed * (2.0 ** -0.5)))


def _dense(temporal, geo, wtT, wgT, b2, ga2, be2, block_b=4096, interpret=False):
  B = temporal.shape[0]
  grid = (B // block_b,)
  out = pl.pallas_call(
      _dense_body,
      grid=grid,
      in_specs=[
          pl.BlockSpec((block_b, TEMPORAL_DIM), lambda i: (i, 0)),
          pl.BlockSpec((block_b, GEO_DIM), lambda i: (i, 0)),
          pl.BlockSpec((OUT_DIM, TEMPORAL_DIM), lambda i: (0, 0)),
          pl.BlockSpec((OUT_DIM, GEO_DIM), lambda i: (0, 0)),
          pl.BlockSpec((OUT_DIM, 1), lambda i: (0, 0)),
          pl.BlockSpec((OUT_DIM, 1), lambda i: (0, 0)),
          pl.BlockSpec((OUT_DIM, 1), lambda i: (0, 0)),
      ],
      out_specs=pl.BlockSpec((OUT_DIM, block_b), lambda i: (0, i)),
      out_shape=jax.ShapeDtypeStruct((OUT_DIM, B), jnp.float32),
      interpret=interpret,
  )(temporal, geo, wtT, wgT, b2, ga2, be2)
  return out.T


def kernel(temporal_features, geohash_buckets, emb_table, W, b, ln_gamma, ln_beta):
  B = temporal_features.shape[0]
  V, D = emb_table.shape
  idx = geohash_buckets.astype(jnp.int32)
  tbl_t = emb_table.T
  tail_start = (V - 1) // SLAB_COLS * SLAB_COLS
  tail_cols = V - tail_start
  tail_pad = (tail_cols + 127) // 128 * 128
  tail = jnp.pad(emb_table[tail_start:].T, ((0, 0), (0, tail_pad - tail_cols)))
  rows, pos = _make_lookup(V, D, B)(tbl_t, tail, idx)
  geo = _make_scatter(D, B)(rows.reshape(-1, D), pos)
  wtT = W[:TEMPORAL_DIM].T
  wgT = W[TEMPORAL_DIM:].T
  return _dense(temporal_features, geo, wtT, wgT,
                b[:, None], ln_gamma[:, None], ln_beta[:, None])


# speedup vs baseline: 1.5010x; 1.0035x over previous
import functools

import jax
import jax.numpy as jnp
from jax import lax
from jax.experimental import pallas as pl
from jax.experimental.pallas import tpu as pltpu
from jax.experimental.pallas import tpu_sc as plsc

GEO_DIM = 16
TEMPORAL_DIM = 4
OUT_DIM = 32

SLAB_COLS = 2048
CAP = 768
LIST = CAP + 16


@functools.lru_cache(maxsize=None)
def _make_lookup(V, D, B):
  info = plsc.get_sparse_core_info()
  NC, NS = info.num_cores, info.num_subcores
  NW = NC * NS
  n_slabs = (V + SLAB_COLS - 1) // SLAB_COLS
  n_rounds = (n_slabs + NW - 1) // NW
  last_slab = n_slabs - 1
  tail_cols = V - last_slab * SLAB_COLS
  n_groups = B // 16
  mesh = plsc.VectorSubcoreMesh(core_axis_name="c", subcore_axis_name="s")

  @functools.partial(
      pl.kernel,
      mesh=mesh,
      compiler_params=pltpu.CompilerParams(
          use_tc_tiling_on_sc=True, needs_layout_passes=False),
      out_type=(jax.ShapeDtypeStruct((NW * CAP * D,), jnp.float32),
                jax.ShapeDtypeStruct((NW * CAP,), jnp.int32)),
      scratch_types=[
          pltpu.VMEM((B,), jnp.int32),
          pltpu.VMEM((LIST,), jnp.int32),
          pltpu.VMEM((LIST,), jnp.int32),
          pltpu.VMEM((LIST,), jnp.int32),
          pltpu.VMEM((LIST,), jnp.int32),
          pltpu.VMEM(((CAP + 16) * D,), jnp.float32),
          pltpu.VMEM((D, SLAB_COLS), jnp.float32),
          pltpu.VMEM((D, SLAB_COLS), jnp.float32),
          pltpu.SemaphoreType.DMA,
      ],
  )
  def lookup_k(table_hbm, tail_hbm, idx_hbm, rows_hbm, pos_hbm,
               idx_v, myidx_v, mypos_v, gidx_v, gpos_v,
               stage_v, slab_a, slab_b, sem):
    w = lax.axis_index("s") * NC + lax.axis_index("c")
    d16 = lax.iota(jnp.int32, 16)
    slabs = (slab_a, slab_b)

    cps = [pltpu.async_copy(table_hbm.at[:, pl.ds(w * SLAB_COLS, SLAB_COLS)],
                            slab_a, sem)]

    pltpu.sync_copy(idx_hbm, idx_v)

    def scan_body(k, cur):
      v = idx_v[pl.ds(k * 16, 16)]
      mine = ((v >> 11) & (NW - 1)) == w
      plsc.store_compressed(myidx_v.at[pl.ds(cur, 16)], v, mask=mine)
      plsc.store_compressed(mypos_v.at[pl.ds(cur, 16)], k * 16 + d16, mask=mine)
      cnt = plsc.all_reduce_population_count(mine)[0]
      return jnp.minimum(cur + cnt, CAP)

    cur = lax.fori_loop(0, n_groups, scan_body, jnp.int32(0), unroll=8)

    for i in range(LIST // 16):
      gpos_v[pl.ds(i * 16, 16)] = jnp.full((16,), B, dtype=jnp.int32)

    n_win = (cur + 15) >> 4
    starts = []
    cur2 = jnp.int32(0)
    for r in range(n_rounds):
      starts.append(cur2)

      def group_body(k, c2, r=r):
        e = myidx_v[pl.ds(k * 16, 16)]
        p = mypos_v[pl.ds(k * 16, 16)]
        m = ((e >> 16) == r) & ((k * 16 + d16) < cur)
        plsc.store_compressed(gidx_v.at[pl.ds(c2, 16)], e, mask=m)
        plsc.store_compressed(gpos_v.at[pl.ds(c2, 16)], p, mask=m)
        cnt = plsc.all_reduce_population_count(m)[0]
        return jnp.minimum(c2 + cnt, CAP)

      cur2 = lax.fori_loop(0, n_win, group_body, cur2)
    starts.append(cur2)

    def gather_round(r, buf, tail):
      base_c = (r * NW + w) * SLAB_COLS

      def gather_body(g, _):
        base_e = starts[r] + g * 16
        colv = gidx_v[pl.ds(base_e, 16)] - base_c
        for j in range(16):
          @pl.when(base_e + j < starts[r + 1])
          def _():
            cj = colv[j]
            row = plsc.load_gather(buf, [d16, jnp.full((16,), cj)])
            plsc.store_scatter(stage_v, [(base_e + j) * D + d16], row)
        return 0

      n_g = (starts[r + 1] - starts[r] + 15) >> 4
      lax.fori_loop(0, n_g, gather_body, 0)

    for r in range(n_rounds - 1):
      cps[r].wait()
      if r + 1 <= n_rounds - 2:
        nxt = (r + 1) * NW + w
        cps.append(
            pltpu.async_copy(table_hbm.at[:, pl.ds(nxt * SLAB_COLS, SLAB_COLS)],
                             slabs[(r + 1) % 2], sem))
      gather_round(r, slabs[r % 2], tail=False)

    last_buf = slabs[(n_rounds - 1) % 2]

    @pl.when(w < (n_slabs - (n_rounds - 1) * NW) - 1)
    def _():
      base_c = ((n_rounds - 1) * NW + w) * SLAB_COLS
      pltpu.sync_copy(table_hbm.at[:, pl.ds(base_c, SLAB_COLS)],
                      last_buf.at[:, pl.ds(0, SLAB_COLS)])
      gather_round(n_rounds - 1, last_buf, tail=False)

    @pl.when(w == (n_slabs - (n_rounds - 1) * NW) - 1)
    def _():
      pltpu.sync_copy(tail_hbm, last_buf.at[:, pl.ds(0, tail_hbm.shape[1])])
      gather_round(n_rounds - 1, last_buf, tail=True)

    pltpu.sync_copy(stage_v.at[pl.ds(0, CAP * D)],
                    rows_hbm.at[pl.ds(w * CAP * D, CAP * D)])
    pltpu.sync_copy(gpos_v.at[pl.ds(0, CAP)], pos_hbm.at[pl.ds(w * CAP, CAP)])

  return lookup_k


@functools.lru_cache(maxsize=None)
def _make_scatter(D, B):
  info = plsc.get_sparse_core_info()
  NC, NS = info.num_cores, info.num_subcores
  NW = NC * NS
  n_win = CAP // 128
  mesh = plsc.VectorSubcoreMesh(core_axis_name="c", subcore_axis_name="s")

  seg = B // NW

  @functools.partial(
      pl.kernel,
      mesh=mesh,
      compiler_params=pltpu.CompilerParams(
          use_tc_tiling_on_sc=False, needs_layout_passes=False),
      out_type=jax.ShapeDtypeStruct((B, D), jnp.float32),
      scratch_types=[
          pltpu.VMEM((NW * CAP,), jnp.int32),
          pltpu.VMEM((seg // 128, 128), jnp.int32),
          pltpu.VMEM((seg, D), jnp.float32),
          pltpu.SemaphoreType.DMA,
      ],
  )
  def unpermute_k(rows_hbm, pos_hbm, out_hbm, pos_v, e2p_v, rows_v, sem):
    w = lax.axis_index("s") * NC + lax.axis_index("c")
    d16 = lax.iota(jnp.int32, 16)
    pltpu.sync_copy(pos_hbm, pos_v)

    def scan_body(k, _):
      p = pos_v[pl.ds(k * 16, 16)]
      m = (p >> 9) == w
      off = p & (seg - 1)
      plsc.store_scatter(e2p_v, [off >> 7, off & 127], k * 16 + d16, mask=m)
      return 0

    lax.fori_loop(0, (NW * CAP) // 16, scan_body, 0, unroll=8)

    copies = []
    for q in range(seg // 128):
      copies.append(
          pltpu.async_copy(rows_hbm.at[e2p_v.at[q]],
                           rows_v.at[pl.ds(q * 128, 128)], sem))
    for c in copies:
      c.wait()
    pltpu.sync_copy(rows_v, out_hbm.at[pl.ds(w * seg, seg)])

  return unpermute_k


def _dense_body(t_ref, g_ref, wt_ref, wg_ref, b_ref, ga_ref, be_ref, o_ref):
  dn = (((1,), (1,)), ((), ()))
  h = lax.dot_general(wt_ref[...], t_ref[...], dn,
                      preferred_element_type=jnp.float32)
  h = h + lax.dot_general(wg_ref[...], g_ref[...], dn,
                          preferred_element_type=jnp.float32)
  h = h + b_ref[...]
  mean = jnp.mean(h, axis=0, keepdims=True)
  c = h - mean
  var = jnp.mean(c * c, axis=0, keepdims=True)
  normed = c * lax.rsqrt(var + 1e-5)
  normed = normed * ga_ref[...] + be_ref[...]
  o_ref[...] = normed * 0.5 * (1.0 + lax.erf(normed * (2.0 ** -0.5)))


def _dense(temporal, geo, wtT, wgT, b2, ga2, be2, block_b=8192, interpret=False):
  B = temporal.shape[0]
  grid = (B // block_b,)
  out = pl.pallas_call(
      _dense_body,
      grid=grid,
      in_specs=[
          pl.BlockSpec((block_b, TEMPORAL_DIM), lambda i: (i, 0)),
          pl.BlockSpec((block_b, GEO_DIM), lambda i: (i, 0)),
          pl.BlockSpec((OUT_DIM, TEMPORAL_DIM), lambda i: (0, 0)),
          pl.BlockSpec((OUT_DIM, GEO_DIM), lambda i: (0, 0)),
          pl.BlockSpec((OUT_DIM, 1), lambda i: (0, 0)),
          pl.BlockSpec((OUT_DIM, 1), lambda i: (0, 0)),
          pl.BlockSpec((OUT_DIM, 1), lambda i: (0, 0)),
      ],
      out_specs=pl.BlockSpec((OUT_DIM, block_b), lambda i: (0, i)),
      out_shape=jax.ShapeDtypeStruct((OUT_DIM, B), jnp.float32),
      interpret=interpret,
  )(temporal, geo, wtT, wgT, b2, ga2, be2)
  return out.T


def kernel(temporal_features, geohash_buckets, emb_table, W, b, ln_gamma, ln_beta):
  B = temporal_features.shape[0]
  V, D = emb_table.shape
  idx = geohash_buckets.astype(jnp.int32)
  tbl_t = emb_table.T
  tail_start = (V - 1) // SLAB_COLS * SLAB_COLS
  tail_cols = V - tail_start
  tail_pad = (tail_cols + 127) // 128 * 128
  tail = jnp.pad(emb_table[tail_start:].T, ((0, 0), (0, tail_pad - tail_cols)))
  rows, pos = _make_lookup(V, D, B)(tbl_t, tail, idx)
  geo = _make_scatter(D, B)(rows.reshape(-1, D), pos)
  wtT = W[:TEMPORAL_DIM].T
  wgT = W[TEMPORAL_DIM:].T
  return _dense(temporal_features, geo, wtT, wgT,
                b[:, None], ln_gamma[:, None], ln_beta[:, None])


# speedup vs baseline: 1.5024x; 1.0009x over previous
import functools

import jax
import jax.numpy as jnp
from jax import lax
from jax.experimental import pallas as pl
from jax.experimental.pallas import tpu as pltpu
from jax.experimental.pallas import tpu_sc as plsc

GEO_DIM = 16
TEMPORAL_DIM = 4
OUT_DIM = 32

SLAB_COLS = 2048
CAP = 768
LIST = CAP + 16


@functools.lru_cache(maxsize=None)
def _make_lookup(V, D, B):
  info = plsc.get_sparse_core_info()
  NC, NS = info.num_cores, info.num_subcores
  NW = NC * NS
  n_slabs = (V + SLAB_COLS - 1) // SLAB_COLS
  n_rounds = (n_slabs + NW - 1) // NW
  last_slab = n_slabs - 1
  tail_cols = V - last_slab * SLAB_COLS
  n_groups = B // 16
  mesh = plsc.VectorSubcoreMesh(core_axis_name="c", subcore_axis_name="s")

  @functools.partial(
      pl.kernel,
      mesh=mesh,
      compiler_params=pltpu.CompilerParams(
          use_tc_tiling_on_sc=True, needs_layout_passes=False),
      out_type=(jax.ShapeDtypeStruct((NW * CAP * D,), jnp.float32),
                jax.ShapeDtypeStruct((NW * CAP,), jnp.int32)),
      scratch_types=[
          pltpu.VMEM((B,), jnp.int32),
          pltpu.VMEM((LIST,), jnp.int32),
          pltpu.VMEM((LIST,), jnp.int32),
          pltpu.VMEM((LIST,), jnp.int32),
          pltpu.VMEM((LIST,), jnp.int32),
          pltpu.VMEM(((CAP + 16) * D,), jnp.float32),
          pltpu.VMEM((D, SLAB_COLS), jnp.float32),
          pltpu.VMEM((D, SLAB_COLS), jnp.float32),
          pltpu.SemaphoreType.DMA,
      ],
  )
  def lookup_k(table_hbm, tail_hbm, idx_hbm, rows_hbm, pos_hbm,
               idx_v, myidx_v, mypos_v, gidx_v, gpos_v,
               stage_v, slab_a, slab_b, sem):
    w = lax.axis_index("s") * NC + lax.axis_index("c")
    d16 = lax.iota(jnp.int32, 16)
    slabs = (slab_a, slab_b)

    cps = [pltpu.async_copy(table_hbm.at[:, pl.ds(w * SLAB_COLS, SLAB_COLS)],
                            slab_a, sem)]

    pltpu.sync_copy(idx_hbm, idx_v)

    def scan_body(k, cur):
      v = idx_v[pl.ds(k * 16, 16)]
      mine = ((v >> 11) & (NW - 1)) == w
      plsc.store_compressed(myidx_v.at[pl.ds(cur, 16)], v, mask=mine)
      plsc.store_compressed(mypos_v.at[pl.ds(cur, 16)], k * 16 + d16, mask=mine)
      cnt = plsc.all_reduce_population_count(mine)[0]
      return jnp.minimum(cur + cnt, CAP)

    cur = lax.fori_loop(0, n_groups, scan_body, jnp.int32(0), unroll=8)

    for i in range(LIST // 16):
      gpos_v[pl.ds(i * 16, 16)] = jnp.full((16,), B, dtype=jnp.int32)

    n_win = (cur + 15) >> 4
    starts = []
    cur2 = jnp.int32(0)
    for r in range(n_rounds):
      starts.append(cur2)

      def group_body(k, c2, r=r):
        e = myidx_v[pl.ds(k * 16, 16)]
        p = mypos_v[pl.ds(k * 16, 16)]
        m = ((e >> 16) == r) & ((k * 16 + d16) < cur)
        plsc.store_compressed(gidx_v.at[pl.ds(c2, 16)], e, mask=m)
        plsc.store_compressed(gpos_v.at[pl.ds(c2, 16)], p, mask=m)
        cnt = plsc.all_reduce_population_count(m)[0]
        return jnp.minimum(c2 + cnt, CAP)

      cur2 = lax.fori_loop(0, n_win, group_body, cur2)
    starts.append(cur2)

    def gather_round(r, buf, tail):
      base_c = (r * NW + w) * SLAB_COLS

      def gather_body(g, _):
        base_e = starts[r] + g * 16
        colv = gidx_v[pl.ds(base_e, 16)] - base_c
        for j in range(16):
          @pl.when(base_e + j < starts[r + 1])
          def _():
            cj = colv[j]
            row = plsc.load_gather(buf, [d16, jnp.full((16,), cj)])
            plsc.store_scatter(stage_v, [(base_e + j) * D + d16], row)
        return 0

      n_g = (starts[r + 1] - starts[r] + 15) >> 4
      lax.fori_loop(0, n_g, gather_body, 0)

    for r in range(n_rounds - 1):
      cps[r].wait()
      if r + 1 <= n_rounds - 2:
        nxt = (r + 1) * NW + w
        cps.append(
            pltpu.async_copy(table_hbm.at[:, pl.ds(nxt * SLAB_COLS, SLAB_COLS)],
                             slabs[(r + 1) % 2], sem))
      gather_round(r, slabs[r % 2], tail=False)

    last_buf = slabs[(n_rounds - 1) % 2]

    @pl.when(w < (n_slabs - (n_rounds - 1) * NW) - 1)
    def _():
      base_c = ((n_rounds - 1) * NW + w) * SLAB_COLS
      pltpu.sync_copy(table_hbm.at[:, pl.ds(base_c, SLAB_COLS)],
                      last_buf.at[:, pl.ds(0, SLAB_COLS)])
      gather_round(n_rounds - 1, last_buf, tail=False)

    @pl.when(w == (n_slabs - (n_rounds - 1) * NW) - 1)
    def _():
      pltpu.sync_copy(tail_hbm, last_buf.at[:, pl.ds(0, tail_hbm.shape[1])])
      gather_round(n_rounds - 1, last_buf, tail=True)

    pltpu.sync_copy(stage_v.at[pl.ds(0, CAP * D)],
                    rows_hbm.at[pl.ds(w * CAP * D, CAP * D)])
    pltpu.sync_copy(gpos_v.at[pl.ds(0, CAP)], pos_hbm.at[pl.ds(w * CAP, CAP)])

  return lookup_k


@functools.lru_cache(maxsize=None)
def _make_scatter(D, B):
  info = plsc.get_sparse_core_info()
  NC, NS = info.num_cores, info.num_subcores
  NW = NC * NS
  n_win = CAP // 128
  mesh = plsc.VectorSubcoreMesh(core_axis_name="c", subcore_axis_name="s")

  seg = B // NW
  assert seg & (seg - 1) == 0
  seg_shift = seg.bit_length() - 1

  @functools.partial(
      pl.kernel,
      mesh=mesh,
      compiler_params=pltpu.CompilerParams(
          use_tc_tiling_on_sc=False, needs_layout_passes=False),
      out_type=jax.ShapeDtypeStruct((B, D), jnp.float32),
      scratch_types=[
          pltpu.VMEM((NW * CAP,), jnp.int32),
          pltpu.VMEM((seg // 128, 128), jnp.int32),
          pltpu.VMEM((seg, D), jnp.float32),
          pltpu.SemaphoreType.DMA,
      ],
  )
  def unpermute_k(rows_hbm, pos_hbm, out_hbm, pos_v, e2p_v, rows_v, sem):
    w = lax.axis_index("s") * NC + lax.axis_index("c")
    d16 = lax.iota(jnp.int32, 16)
    pltpu.sync_copy(pos_hbm, pos_v)

    def scan_body(k, _):
      p = pos_v[pl.ds(k * 16, 16)]
      m = (p >> seg_shift) == w
      off = p & (seg - 1)
      plsc.store_scatter(e2p_v, [off >> 7, off & 127], k * 16 + d16, mask=m)
      return 0

    lax.fori_loop(0, (NW * CAP) // 16, scan_body, 0, unroll=8)

    copies = []
    for q in range(seg // 128):
      copies.append(
          pltpu.async_copy(rows_hbm.at[e2p_v.at[q]],
                           rows_v.at[pl.ds(q * 128, 128)], sem))
    for c in copies:
      c.wait()
    pltpu.sync_copy(rows_v, out_hbm.at[pl.ds(w * seg, seg)])

  return unpermute_k


def _dense_body(t_ref, g_ref, wt_ref, wg_ref, b_ref, ga_ref, be_ref, o_ref):
  dn = (((1,), (1,)), ((), ()))
  h = lax.dot_general(wt_ref[...], t_ref[...], dn,
                      preferred_element_type=jnp.float32)
  h = h + lax.dot_general(wg_ref[...], g_ref[...], dn,
                          preferred_element_type=jnp.float32)
  h = h + b_ref[...]
  mean = jnp.mean(h, axis=0, keepdims=True)
  c = h - mean
  var = jnp.mean(c * c, axis=0, keepdims=True)
  normed = c * lax.rsqrt(var + 1e-5)
  normed = normed * ga_ref[...] + be_ref[...]
  o_ref[...] = normed * 0.5 * (1.0 + lax.erf(normed * (2.0 ** -0.5)))


def _dense(temporal, geo, wtT, wgT, b2, ga2, be2, block_b=8192, interpret=False):
  B = temporal.shape[0]
  grid = (B // block_b,)
  out = pl.pallas_call(
      _dense_body,
      grid=grid,
      in_specs=[
          pl.BlockSpec((block_b, TEMPORAL_DIM), lambda i: (i, 0)),
          pl.BlockSpec((block_b, GEO_DIM), lambda i: (i, 0)),
          pl.BlockSpec((OUT_DIM, TEMPORAL_DIM), lambda i: (0, 0)),
          pl.BlockSpec((OUT_DIM, GEO_DIM), lambda i: (0, 0)),
          pl.BlockSpec((OUT_DIM, 1), lambda i: (0, 0)),
          pl.BlockSpec((OUT_DIM, 1), lambda i: (0, 0)),
          pl.BlockSpec((OUT_DIM, 1), lambda i: (0, 0)),
      ],
      out_specs=pl.BlockSpec((OUT_DIM, block_b), lambda i: (0, i)),
      out_shape=jax.ShapeDtypeStruct((OUT_DIM, B), jnp.float32),
      interpret=interpret,
  )(temporal, geo, wtT, wgT, b2, ga2, be2)
  return out.T


def kernel(temporal_features, geohash_buckets, emb_table, W, b, ln_gamma, ln_beta):
  B = temporal_features.shape[0]
  V, D = emb_table.shape
  idx = geohash_buckets.astype(jnp.int32)
  tbl_t = emb_table.T
  tail_start = (V - 1) // SLAB_COLS * SLAB_COLS
  tail_cols = V - tail_start
  tail_pad = (tail_cols + 127) // 128 * 128
  tail = jnp.pad(emb_table[tail_start:].T, ((0, 0), (0, tail_pad - tail_cols)))
  rows, pos = _make_lookup(V, D, B)(tbl_t, tail, idx)
  geo = _make_scatter(D, B)(rows.reshape(-1, D), pos)
  wtT = W[:TEMPORAL_DIM].T
  wgT = W[TEMPORAL_DIM:].T
  return _dense(temporal_features, geo, wtT, wgT,
                b[:, None], ln_gamma[:, None], ln_beta[:, None])
